# baseline (speedup 1.0000x reference)
import jax
import jax.numpy as jnp
from jax.experimental import pallas as pl
from jax.experimental.pallas import tpu as pltpu

BTILE = 256
NOUT_PAD = 16


def _lenet_kernel(x_ref, w1_ref, b1_ref, w2_ref, b2_ref,
                  w1p_ref, b1p_ref, w2p_ref, b2p_ref, w3p_ref, b3p_ref,
                  o_ref, ps_ref, f_ref):
    slabs1 = {}

    def slab1(c, d):
        if (c, d) not in slabs1:
            slabs1[(c, d)] = x_ref[(c % 4) * 4 + (d % 4),
                                   (c // 4):(c // 4) + 8,
                                   (d // 4):(d // 4) + 8, :]
        return slabs1[(c, d)]

    for ch in range(6):
        for al in range(2):
            for be in range(2):
                best = None
                for a in range(2):
                    for b in range(2):
                        acc = None
                        for dh in range(3):
                            for dw in range(3):
                                t = slab1(2 * al + a + dh, 2 * be + b + dw) \
                                    * w1_ref[ch, dh * 3 + dw]
                                acc = t if acc is None else acc + t
                        best = acc if best is None else jnp.maximum(best, acc)
                ps_ref[ch, al, be] = jnp.maximum(best + b1_ref[ch], 0.0)

    slabs2 = {}

    def slab2(c1, c, d):
        if (c1, c, d) not in slabs2:
            slabs2[(c1, c, d)] = ps_ref[c1, c % 2, d % 2,
                                        (c // 2):(c // 2) + 7,
                                        (d // 2):(d // 2) + 7, :]
        return slabs2[(c1, c, d)]

    zcol = jnp.zeros((7, 1, o_ref.shape[-1]), jnp.float32)
    for c2 in range(16):
        best = None
        for g in range(2):
            for h in range(2):
                acc = None
                for c1 in range(6):
                    for eh in range(2):
                        for ew in range(2):
                            t = slab2(c1, g + eh, h + ew) \
                                * w2_ref[c2, c1 * 4 + eh * 2 + ew]
                            acc = t if acc is None else acc + t
                best = acc if best is None else jnp.maximum(best, acc)
        p2 = jnp.maximum(best + b2_ref[c2], 0.0)
        f_ref[c2] = jnp.concatenate([p2, zcol], axis=1)

    feats = f_ref[...].reshape(16 * 7 * 8, o_ref.shape[-1])
    h = jnp.dot(w1p_ref[...], feats, preferred_element_type=jnp.float32)
    h = jnp.maximum(h + b1p_ref[...], 0.0)
    h = jnp.dot(w2p_ref[...], h, preferred_element_type=jnp.float32)
    h = jnp.maximum(h + b2p_ref[...], 0.0)
    out = jnp.dot(w3p_ref[...], h, preferred_element_type=jnp.float32)
    o_ref[...] = (out + b3p_ref[...]).astype(o_ref.dtype)


def _forward(x4, c1w, c1b, c2w, c2b, w1p, b1p, w2p, b2p, w3p, b3p):
    n_pad = x4.shape[-1]
    nt = n_pad // BTILE
    flops = n_pad * (2 * 6 * 9 * 30 * 30 + 2 * 16 * 24 * 14 * 14
                     + 2 * (896 * 128 + 128 * 128 + 128 * NOUT_PAD))
    bytes_accessed = 4 * (16 * 9 * 16 * n_pad + w1p.size + w2p.size
                          + w3p.size + NOUT_PAD * n_pad)
    smem = pl.BlockSpec(memory_space=pltpu.MemorySpace.SMEM)
    return pl.pallas_call(
        _lenet_kernel,
        out_shape=jax.ShapeDtypeStruct((NOUT_PAD, n_pad), jnp.float32),
        grid=(nt,),
        in_specs=[
            pl.BlockSpec((16, 9, 9, BTILE), lambda i: (0, 0, 0, i)),
            smem, smem, smem, smem,
            pl.BlockSpec((128, 896), lambda i: (0, 0)),
            pl.BlockSpec((128, 1), lambda i: (0, 0)),
            pl.BlockSpec((128, 128), lambda i: (0, 0)),
            pl.BlockSpec((128, 1), lambda i: (0, 0)),
            pl.BlockSpec((NOUT_PAD, 128), lambda i: (0, 0)),
            pl.BlockSpec((NOUT_PAD, 1), lambda i: (0, 0)),
        ],
        out_specs=pl.BlockSpec((NOUT_PAD, BTILE), lambda i: (0, i)),
        scratch_shapes=[
            pltpu.VMEM((6, 2, 2, 8, 8, BTILE), jnp.float32),
            pltpu.VMEM((16, 7, 8, BTILE), jnp.float32),
        ],
        compiler_params=pltpu.CompilerParams(
            dimension_semantics=("parallel",),
            vmem_limit_bytes=64 * 1024 * 1024),
        cost_estimate=pl.CostEstimate(flops=flops, transcendentals=0,
                                      bytes_accessed=bytes_accessed),
    )(x4, c1w, c1b, c2w, c2b, w1p, b1p, w2p, b2p, w3p, b3p)


def kernel(x, conv1_w, conv1_b, conv2_w, conv2_b,
           fc1_w, fc1_b, fc2_w, fc2_b, fc3_w, fc3_b):
    n = x.shape[0]
    n_pad = ((n + BTILE - 1) // BTILE) * BTILE
    xp = jnp.pad(x[:, 0].astype(jnp.float32),
                 ((0, n_pad - n), (2, 6), (2, 6)))
    x4 = (xp.reshape(n_pad, 9, 4, 9, 4)
            .transpose(2, 4, 1, 3, 0)
            .reshape(16, 9, 9, n_pad))
    out = _forward(x4, conv1_w, conv1_b, conv2_w, conv2_b,
                   fc1_w, fc1_b, fc2_w, fc2_b, fc3_w, fc3_b)
    return out[:10, :n].T


# speedup vs baseline: 1.2474x; 1.2474x over previous
import jax
import jax.numpy as jnp
from jax.experimental import pallas as pl
from jax.experimental.pallas import tpu as pltpu

BTILE = 256
NOUT_PAD = 16


def _lenet_kernel(x_ref, w1_ref, b1_ref, w2_ref, b2_ref,
                  w1p_ref, b1p_ref, w2p_ref, b2p_ref, w3p_ref, b3p_ref,
                  o_ref, ps_ref, f_ref):
    slabs1 = {}

    def slab1(c, d):
        if (c, d) not in slabs1:
            cp, cq = (c - 2) % 4, (c - 2) // 4
            dp, dq = (d - 2) % 4, (d - 2) // 4
            base = x_ref[cp * 4 + dp]
            rlo = -cq
            clo = -dq
            slabs1[(c, d)] = jnp.pad(
                base, ((rlo, 1 - rlo), (clo, 1 - clo), (0, 0)))
        return slabs1[(c, d)]

    for ch in range(6):
        for al in range(2):
            for be in range(2):
                best = None
                for a in range(2):
                    for b in range(2):
                        acc = None
                        for dh in range(3):
                            for dw in range(3):
                                t = slab1(2 * al + a + dh, 2 * be + b + dw) \
                                    * w1_ref[ch, dh * 3 + dw]
                                acc = t if acc is None else acc + t
                        best = acc if best is None else jnp.maximum(best, acc)
                ps_ref[ch, al, be] = jnp.maximum(best + b1_ref[ch], 0.0)

    slabs2 = {}

    def slab2(c1, c, d):
        if (c1, c, d) not in slabs2:
            slabs2[(c1, c, d)] = ps_ref[c1, c % 2, d % 2,
                                        (c // 2):(c // 2) + 7,
                                        (d // 2):(d // 2) + 7, :]
        return slabs2[(c1, c, d)]

    zcol = jnp.zeros((7, 1, o_ref.shape[-1]), jnp.float32)
    for c2 in range(16):
        best = None
        for g in range(2):
            for h in range(2):
                acc = None
                for c1 in range(6):
                    for eh in range(2):
                        for ew in range(2):
                            t = slab2(c1, g + eh, h + ew) \
                                * w2_ref[c2, c1 * 4 + eh * 2 + ew]
                            acc = t if acc is None else acc + t
                best = acc if best is None else jnp.maximum(best, acc)
        p2 = jnp.maximum(best + b2_ref[c2], 0.0)
        f_ref[c2] = jnp.concatenate([p2, zcol], axis=1)

    feats = f_ref[...].reshape(16 * 7 * 8, o_ref.shape[-1])
    h = jnp.dot(w1p_ref[...], feats, preferred_element_type=jnp.float32)
    h = jnp.maximum(h + b1p_ref[...], 0.0)
    h = jnp.dot(w2p_ref[...], h, preferred_element_type=jnp.float32)
    h = jnp.maximum(h + b2p_ref[...], 0.0)
    out = jnp.dot(w3p_ref[...], h, preferred_element_type=jnp.float32)
    o_ref[...] = (out + b3p_ref[...]).astype(o_ref.dtype)


def _forward(x4, c1w, c1b, c2w, c2b, w1p, b1p, w2p, b2p, w3p, b3p):
    n_pad = x4.shape[-1]
    nt = n_pad // BTILE
    flops = n_pad * (2 * 6 * 9 * 30 * 30 + 2 * 16 * 24 * 14 * 14
                     + 2 * (896 * 128 + 128 * 128 + 128 * NOUT_PAD))
    bytes_accessed = 4 * (16 * 7 * 7 * n_pad + w1p.size + w2p.size
                          + w3p.size + NOUT_PAD * n_pad)
    smem = pl.BlockSpec(memory_space=pltpu.MemorySpace.SMEM)
    return pl.pallas_call(
        _lenet_kernel,
        out_shape=jax.ShapeDtypeStruct((NOUT_PAD, n_pad), jnp.float32),
        grid=(nt,),
        in_specs=[
            pl.BlockSpec((16, 7, 7, BTILE), lambda i: (0, 0, 0, i)),
            smem, smem, smem, smem,
            pl.BlockSpec((128, 896), lambda i: (0, 0)),
            pl.BlockSpec((128, 1), lambda i: (0, 0)),
            pl.BlockSpec((128, 128), lambda i: (0, 0)),
            pl.BlockSpec((128, 1), lambda i: (0, 0)),
            pl.BlockSpec((NOUT_PAD, 128), lambda i: (0, 0)),
            pl.BlockSpec((NOUT_PAD, 1), lambda i: (0, 0)),
        ],
        out_specs=pl.BlockSpec((NOUT_PAD, BTILE), lambda i: (0, i)),
        scratch_shapes=[
            pltpu.VMEM((6, 2, 2, 8, 8, BTILE), jnp.float32),
            pltpu.VMEM((16, 7, 8, BTILE), jnp.float32),
        ],
        compiler_params=pltpu.CompilerParams(
            dimension_semantics=("parallel",),
            vmem_limit_bytes=64 * 1024 * 1024),
        cost_estimate=pl.CostEstimate(flops=flops, transcendentals=0,
                                      bytes_accessed=bytes_accessed),
    )(x4, c1w, c1b, c2w, c2b, w1p, b1p, w2p, b2p, w3p, b3p)


def kernel(x, conv1_w, conv1_b, conv2_w, conv2_b,
           fc1_w, fc1_b, fc2_w, fc2_b, fc3_w, fc3_b):
    n = x.shape[0]
    n_pad = ((n + BTILE - 1) // BTILE) * BTILE
    xb = x[:, 0].astype(jnp.float32)
    if n_pad != n:
        xb = jnp.pad(xb, ((0, n_pad - n), (0, 0), (0, 0)))
    x4 = (xb.reshape(n_pad, 7, 4, 7, 4)
            .transpose(2, 4, 1, 3, 0)
            .reshape(16, 7, 7, n_pad))
    out = _forward(x4, conv1_w, conv1_b, conv2_w, conv2_b,
                   fc1_w, fc1_b, fc2_w, fc2_b, fc3_w, fc3_b)
    return out[:10, :n].T


# speedup vs baseline: 1.7376x; 1.3929x over previous
import jax
import jax.numpy as jnp
from jax.experimental import pallas as pl
from jax.experimental.pallas import tpu as pltpu

BTILE = 256
NOUT_PAD = 16


def _lenet_kernel(x_ref, w1_ref, b1_ref, w2r_ref, b2c_ref,
                  w1p_ref, b1p_ref, w2p_ref, b2p_ref, w3p_ref, b3p_ref,
                  o_ref, ps_ref, pb_ref, f2_ref):
    B = o_ref.shape[-1]

    xt = jnp.transpose(x_ref[...], (1, 0))
    xtv = xt.reshape(112, 8, B)

    zrow = jnp.zeros((8, B), jnp.float32)
    slabs1 = {}

    def slab1(c, d):
        if (c, d) not in slabs1:
            d4, dq = (d - 2) % 4, (d - 2) // 4
            rows = []
            for ii in range(8):
                t = 4 * ii + c - 2
                if 0 <= t < 28:
                    r = xtv[4 * t + d4]
                    if dq == -1:
                        r = jnp.pad(r[0:7, :], ((1, 0), (0, 0)))
                    rows.append(r)
                else:
                    rows.append(zrow)
            slabs1[(c, d)] = jnp.stack(rows)
        return slabs1[(c, d)]

    for ch in range(6):
        for al in range(2):
            for be in range(2):
                best = None
                for a in range(2):
                    for b in range(2):
                        acc = None
                        for dh in range(3):
                            for dw in range(3):
                                t = slab1(2 * al + a + dh, 2 * be + b + dw) \
                                    * w1_ref[ch, dh * 3 + dw]
                                acc = t if acc is None else acc + t
                        best = acc if best is None else jnp.maximum(best, acc)
                ps_ref[ch, al, be] = jnp.maximum(best + b1_ref[ch], 0.0)

    for al in range(2):
        for be in range(2):
            for ii in range(8):
                chunk = ps_ref[:, al, be, ii]
                tile = jnp.transpose(chunk, (1, 0, 2))
                tile = jnp.pad(tile, ((0, 0), (0, 2), (0, 0)))
                pb_ref[al * 128 + be * 64 + ii * 8:
                       al * 128 + be * 64 + ii * 8 + 8] = tile

    def pix(i, j):
        return (i % 2) * 128 + (j % 2) * 64 + (i // 2) * 8 + (j // 2)

    w2r = w2r_ref[...]
    b2c = b2c_ref[...]
    for m in range(7):
        for n in range(7):
            best = None
            for g in range(2):
                for h in range(2):
                    v, w = 2 * m + g, 2 * n + h
                    rhs = jnp.stack(
                        [pb_ref[pix(v, w)], pb_ref[pix(v, w + 1)],
                         pb_ref[pix(v + 1, w)], pb_ref[pix(v + 1, w + 1)]]
                    ).reshape(32, B)
                    z = jnp.dot(w2r, rhs,
                                preferred_element_type=jnp.float32)
                    best = z if best is None else jnp.maximum(best, z)
            f2_ref[m * 7 + n] = jnp.maximum(best + b2c, 0.0)

    feats = f2_ref[...].reshape(49 * 16, B)
    h = jnp.dot(w1p_ref[...], feats, preferred_element_type=jnp.float32)
    h = jnp.maximum(h + b1p_ref[...], 0.0)
    h = jnp.dot(w2p_ref[...], h, preferred_element_type=jnp.float32)
    h = jnp.maximum(h + b2p_ref[...], 0.0)
    out = jnp.dot(w3p_ref[...], h, preferred_element_type=jnp.float32)
    o_ref[...] = (out + b3p_ref[...]).astype(o_ref.dtype)


def _forward(xc, c1w, c1b, w2r, b2c, w1p, b1p, w2p, b2p, w3p, b3p):
    n_pad = xc.shape[0]
    nt = n_pad // BTILE
    flops = n_pad * (2 * 6 * 9 * 30 * 30 + 2 * 16 * 32 * 14 * 14
                     + 2 * (784 * 128 + 128 * 128 + 128 * NOUT_PAD))
    bytes_accessed = 4 * (896 * n_pad + w1p.size + w2p.size
                          + w3p.size + NOUT_PAD * n_pad)
    smem = pl.BlockSpec(memory_space=pltpu.MemorySpace.SMEM)
    return pl.pallas_call(
        _lenet_kernel,
        out_shape=jax.ShapeDtypeStruct((NOUT_PAD, n_pad), jnp.float32),
        grid=(nt,),
        in_specs=[
            pl.BlockSpec((BTILE, 896), lambda i: (i, 0)),
            smem, smem,
            pl.BlockSpec((16, 32), lambda i: (0, 0)),
            pl.BlockSpec((16, 1), lambda i: (0, 0)),
            pl.BlockSpec((128, 784), lambda i: (0, 0)),
            pl.BlockSpec((128, 1), lambda i: (0, 0)),
            pl.BlockSpec((128, 128), lambda i: (0, 0)),
            pl.BlockSpec((128, 1), lambda i: (0, 0)),
            pl.BlockSpec((NOUT_PAD, 128), lambda i: (0, 0)),
            pl.BlockSpec((NOUT_PAD, 1), lambda i: (0, 0)),
        ],
        out_specs=pl.BlockSpec((NOUT_PAD, BTILE), lambda i: (0, i)),
        scratch_shapes=[
            pltpu.VMEM((6, 2, 2, 8, 8, BTILE), jnp.float32),
            pltpu.VMEM((256, 8, BTILE), jnp.float32),
            pltpu.VMEM((49, 16, BTILE), jnp.float32),
        ],
        compiler_params=pltpu.CompilerParams(
            dimension_semantics=("parallel",),
            vmem_limit_bytes=64 * 1024 * 1024),
        cost_estimate=pl.CostEstimate(flops=flops, transcendentals=0,
                                      bytes_accessed=bytes_accessed),
    )(xc, c1w, c1b, w2r, b2c, w1p, b1p, w2p, b2p, w3p, b3p)


def kernel(x, conv1_w, conv1_b, conv2_w, conv2_b,
           fc1_w, fc1_b, fc2_w, fc2_b, fc3_w, fc3_b):
    n = x.shape[0]
    n_pad = ((n + BTILE - 1) // BTILE) * BTILE
    xb = x[:, 0].astype(jnp.float32).reshape(n, 28, 7, 4)
    xc = jnp.pad(xb.transpose(0, 1, 3, 2),
                 ((0, n_pad - n), (0, 0), (0, 0), (0, 1))).reshape(n_pad, 896)
    w2r = jnp.pad(conv2_w.reshape(16, 6, 4).transpose(0, 2, 1),
                  ((0, 0), (0, 0), (0, 2))).reshape(16, 32)
    b2c = conv2_b.reshape(16, 1)
    fc1r = (fc1_w.reshape(128, 16, 7, 8)[:, :, :, :7]
            .transpose(0, 2, 3, 1).reshape(128, 784))
    out = _forward(xc, conv1_w, conv1_b, w2r, b2c,
                   fc1r, fc1_b, fc2_w, fc2_b, fc3_w, fc3_b)
    return out[:10, :n].T


# speedup vs baseline: 2.0554x; 1.1829x over previous
import jax
import jax.numpy as jnp
from jax.experimental import pallas as pl
from jax.experimental.pallas import tpu as pltpu

BTILE = 256
NOUT_PAD = 16


def _lenet_kernel(x_ref, w1_ref, b1_ref, w2r_ref, b2c_ref,
                  w1p_ref, b1p_ref, w2p_ref, b2p_ref, w3p_ref, b3p_ref,
                  o_ref, ps_ref, pb_ref, f2_ref):
    B = o_ref.shape[-1]

    xt = jnp.transpose(x_ref[...], (1, 0))
    xr = xt.reshape(98, 8, B)
    zlane = jnp.zeros((1, B), jnp.float32)

    rows4 = {}

    def prow(t, d4):
        if (t, d4) not in rows4:
            q = 28 * t + d4
            srcs = [xr[(q + 4 * s) // 8, (q + 4 * s) % 8].reshape(1, B)
                    for s in range(7)]
            rows4[(t, d4)] = jnp.concatenate(srcs + [zlane])
        return rows4[(t, d4)]

    zrow = jnp.zeros((8, B), jnp.float32)
    slabs1 = {}

    def slab1(c, d):
        if (c, d) not in slabs1:
            d4, dq = (d - 2) % 4, (d - 2) // 4
            rows = []
            for ii in range(8):
                t = 4 * ii + c - 2
                if 0 <= t < 28:
                    r = prow(t, d4)
                    if dq == -1:
                        r = jnp.pad(r[0:7, :], ((1, 0), (0, 0)))
                    rows.append(r)
                else:
                    rows.append(zrow)
            slabs1[(c, d)] = jnp.stack(rows)
        return slabs1[(c, d)]

    for ch in range(6):
        for al in range(2):
            for be in range(2):
                best = None
                for a in range(2):
                    for b in range(2):
                        acc = None
                        for dh in range(3):
                            for dw in range(3):
                                t = slab1(2 * al + a + dh, 2 * be + b + dw) \
                                    * w1_ref[ch, dh * 3 + dw]
                                acc = t if acc is None else acc + t
                        best = acc if best is None else jnp.maximum(best, acc)
                ps_ref[ch, al, be] = jnp.maximum(best + b1_ref[ch], 0.0)

    for al in range(2):
        for be in range(2):
            for ii in range(8):
                chunk = ps_ref[:, al, be, ii]
                tile = jnp.transpose(chunk, (1, 0, 2))
                tile = jnp.pad(tile, ((0, 0), (0, 2), (0, 0)))
                pb_ref[al * 128 + be * 64 + ii * 8:
                       al * 128 + be * 64 + ii * 8 + 8] = tile

    def pix(i, j):
        return (i % 2) * 128 + (j % 2) * 64 + (i // 2) * 8 + (j // 2)

    w2r = w2r_ref[...]
    b2c = b2c_ref[...]
    for m in range(7):
        for n in range(7):
            best = None
            for g in range(2):
                for h in range(2):
                    v, w = 2 * m + g, 2 * n + h
                    rhs = jnp.stack(
                        [pb_ref[pix(v, w)], pb_ref[pix(v, w + 1)],
                         pb_ref[pix(v + 1, w)], pb_ref[pix(v + 1, w + 1)]]
                    ).reshape(32, B)
                    z = jnp.dot(w2r, rhs,
                                preferred_element_type=jnp.float32)
                    best = z if best is None else jnp.maximum(best, z)
            f2_ref[m * 7 + n] = jnp.maximum(best + b2c, 0.0)

    feats = f2_ref[...].reshape(49 * 16, B)
    h = jnp.dot(w1p_ref[...], feats, preferred_element_type=jnp.float32)
    h = jnp.maximum(h + b1p_ref[...], 0.0)
    h = jnp.dot(w2p_ref[...], h, preferred_element_type=jnp.float32)
    h = jnp.maximum(h + b2p_ref[...], 0.0)
    out = jnp.dot(w3p_ref[...], h, preferred_element_type=jnp.float32)
    o_ref[...] = (out + b3p_ref[...]).astype(o_ref.dtype)


def _forward(xc, c1w, c1b, w2r, b2c, w1p, b1p, w2p, b2p, w3p, b3p):
    n_pad = xc.shape[0]
    nt = n_pad // BTILE
    flops = n_pad * (2 * 6 * 9 * 30 * 30 + 2 * 16 * 32 * 14 * 14
                     + 2 * (784 * 128 + 128 * 128 + 128 * NOUT_PAD))
    bytes_accessed = 4 * (896 * n_pad + w1p.size + w2p.size
                          + w3p.size + NOUT_PAD * n_pad)
    smem = pl.BlockSpec(memory_space=pltpu.MemorySpace.SMEM)
    return pl.pallas_call(
        _lenet_kernel,
        out_shape=jax.ShapeDtypeStruct((NOUT_PAD, n_pad), jnp.float32),
        grid=(nt,),
        in_specs=[
            pl.BlockSpec((BTILE, 784), lambda i: (i, 0)),
            smem, smem,
            pl.BlockSpec((16, 32), lambda i: (0, 0)),
            pl.BlockSpec((16, 1), lambda i: (0, 0)),
            pl.BlockSpec((128, 784), lambda i: (0, 0)),
            pl.BlockSpec((128, 1), lambda i: (0, 0)),
            pl.BlockSpec((128, 128), lambda i: (0, 0)),
            pl.BlockSpec((128, 1), lambda i: (0, 0)),
            pl.BlockSpec((NOUT_PAD, 128), lambda i: (0, 0)),
            pl.BlockSpec((NOUT_PAD, 1), lambda i: (0, 0)),
        ],
        out_specs=pl.BlockSpec((NOUT_PAD, BTILE), lambda i: (0, i)),
        scratch_shapes=[
            pltpu.VMEM((6, 2, 2, 8, 8, BTILE), jnp.float32),
            pltpu.VMEM((256, 8, BTILE), jnp.float32),
            pltpu.VMEM((49, 16, BTILE), jnp.float32),
        ],
        compiler_params=pltpu.CompilerParams(
            dimension_semantics=("parallel",),
            vmem_limit_bytes=64 * 1024 * 1024),
        cost_estimate=pl.CostEstimate(flops=flops, transcendentals=0,
                                      bytes_accessed=bytes_accessed),
    )(xc, c1w, c1b, w2r, b2c, w1p, b1p, w2p, b2p, w3p, b3p)


def kernel(x, conv1_w, conv1_b, conv2_w, conv2_b,
           fc1_w, fc1_b, fc2_w, fc2_b, fc3_w, fc3_b):
    n = x.shape[0]
    n_pad = ((n + BTILE - 1) // BTILE) * BTILE
    xc = x.astype(jnp.float32).reshape(n, 784)
    if n_pad != n:
        xc = jnp.pad(xc, ((0, n_pad - n), (0, 0)))
    w2r = jnp.pad(conv2_w.reshape(16, 6, 4).transpose(0, 2, 1),
                  ((0, 0), (0, 0), (0, 2))).reshape(16, 32)
    b2c = conv2_b.reshape(16, 1)
    fc1r = (fc1_w.reshape(128, 16, 7, 8)[:, :, :, :7]
            .transpose(0, 2, 3, 1).reshape(128, 784))
    out = _forward(xc, conv1_w, conv1_b, w2r, b2c,
                   fc1r, fc1_b, fc2_w, fc2_b, fc3_w, fc3_b)
    return out[:10, :n].T


# speedup vs baseline: 2.1039x; 1.0236x over previous
import jax
import jax.numpy as jnp
from jax.experimental import pallas as pl
from jax.experimental.pallas import tpu as pltpu

BTILE = 512
NOUT_PAD = 16


def _lenet_kernel(x_ref, w1_ref, b1_ref, w2r_ref, b2c_ref,
                  w1p_ref, b1p_ref, w2p_ref, b2p_ref, w3p_ref, b3p_ref,
                  o_ref, ps_ref, pb_ref, f2_ref):
    B = o_ref.shape[-1]

    xt = jnp.transpose(x_ref[...], (1, 0))
    xr = xt.reshape(98, 8, B)
    zlane = jnp.zeros((1, B), jnp.float32)

    rows4 = {}

    def prow(t, d4):
        if (t, d4) not in rows4:
            q = 28 * t + d4
            srcs = [xr[(q + 4 * s) // 8, (q + 4 * s) % 8].reshape(1, B)
                    for s in range(7)]
            rows4[(t, d4)] = jnp.concatenate(srcs + [zlane])
        return rows4[(t, d4)]

    zrow = jnp.zeros((8, B), jnp.float32)
    slabs1 = {}

    def slab1(c, d):
        if (c, d) not in slabs1:
            d4, dq = (d - 2) % 4, (d - 2) // 4
            rows = []
            for ii in range(8):
                t = 4 * ii + c - 2
                if 0 <= t < 28:
                    r = prow(t, d4)
                    if dq == -1:
                        r = jnp.pad(r[0:7, :], ((1, 0), (0, 0)))
                    rows.append(r)
                else:
                    rows.append(zrow)
            slabs1[(c, d)] = jnp.stack(rows)
        return slabs1[(c, d)]

    for ch in range(6):
        for al in range(2):
            for be in range(2):
                best = None
                for a in range(2):
                    for b in range(2):
                        acc = None
                        for dh in range(3):
                            for dw in range(3):
                                t = slab1(2 * al + a + dh, 2 * be + b + dw) \
                                    * w1_ref[ch, dh * 3 + dw]
                                acc = t if acc is None else acc + t
                        best = acc if best is None else jnp.maximum(best, acc)
                ps_ref[ch, al, be] = jnp.maximum(best + b1_ref[ch], 0.0)

    for al in range(2):
        for be in range(2):
            for ii in range(8):
                chunk = ps_ref[:, al, be, ii]
                tile = jnp.transpose(chunk, (1, 0, 2))
                tile = jnp.pad(tile, ((0, 0), (0, 2), (0, 0)))
                pb_ref[al * 128 + be * 64 + ii * 8:
                       al * 128 + be * 64 + ii * 8 + 8] = tile

    def pix(i, j):
        return (i % 2) * 128 + (j % 2) * 64 + (i // 2) * 8 + (j // 2)

    w2r = w2r_ref[...]
    b2c = b2c_ref[...]
    for m in range(7):
        for n in range(7):
            best = None
            for g in range(2):
                for h in range(2):
                    v, w = 2 * m + g, 2 * n + h
                    rhs = jnp.stack(
                        [pb_ref[pix(v, w)], pb_ref[pix(v, w + 1)],
                         pb_ref[pix(v + 1, w)], pb_ref[pix(v + 1, w + 1)]]
                    ).reshape(32, B)
                    z = jnp.dot(w2r, rhs,
                                preferred_element_type=jnp.float32)
                    best = z if best is None else jnp.maximum(best, z)
            f2_ref[m * 7 + n] = jnp.maximum(best + b2c, 0.0)

    feats = f2_ref[...].reshape(49 * 16, B)
    h = jnp.dot(w1p_ref[...], feats, preferred_element_type=jnp.float32)
    h = jnp.maximum(h + b1p_ref[...], 0.0)
    h = jnp.dot(w2p_ref[...], h, preferred_element_type=jnp.float32)
    h = jnp.maximum(h + b2p_ref[...], 0.0)
    out = jnp.dot(w3p_ref[...], h, preferred_element_type=jnp.float32)
    o_ref[...] = (out + b3p_ref[...]).astype(o_ref.dtype)


def _forward(xc, c1w, c1b, w2r, b2c, w1p, b1p, w2p, b2p, w3p, b3p):
    n_pad = xc.shape[0]
    nt = n_pad // BTILE
    flops = n_pad * (2 * 6 * 9 * 30 * 30 + 2 * 16 * 32 * 14 * 14
                     + 2 * (784 * 128 + 128 * 128 + 128 * NOUT_PAD))
    bytes_accessed = 4 * (896 * n_pad + w1p.size + w2p.size
                          + w3p.size + NOUT_PAD * n_pad)
    smem = pl.BlockSpec(memory_space=pltpu.MemorySpace.SMEM)
    return pl.pallas_call(
        _lenet_kernel,
        out_shape=jax.ShapeDtypeStruct((NOUT_PAD, n_pad), jnp.float32),
        grid=(nt,),
        in_specs=[
            pl.BlockSpec((BTILE, 784), lambda i: (i, 0)),
            smem, smem,
            pl.BlockSpec((16, 32), lambda i: (0, 0)),
            pl.BlockSpec((16, 1), lambda i: (0, 0)),
            pl.BlockSpec((128, 784), lambda i: (0, 0)),
            pl.BlockSpec((128, 1), lambda i: (0, 0)),
            pl.BlockSpec((128, 128), lambda i: (0, 0)),
            pl.BlockSpec((128, 1), lambda i: (0, 0)),
            pl.BlockSpec((NOUT_PAD, 128), lambda i: (0, 0)),
            pl.BlockSpec((NOUT_PAD, 1), lambda i: (0, 0)),
        ],
        out_specs=pl.BlockSpec((NOUT_PAD, BTILE), lambda i: (0, i)),
        scratch_shapes=[
            pltpu.VMEM((6, 2, 2, 8, 8, BTILE), jnp.float32),
            pltpu.VMEM((256, 8, BTILE), jnp.float32),
            pltpu.VMEM((49, 16, BTILE), jnp.float32),
        ],
        compiler_params=pltpu.CompilerParams(
            dimension_semantics=("parallel",),
            vmem_limit_bytes=64 * 1024 * 1024),
        cost_estimate=pl.CostEstimate(flops=flops, transcendentals=0,
                                      bytes_accessed=bytes_accessed),
    )(xc, c1w, c1b, w2r, b2c, w1p, b1p, w2p, b2p, w3p, b3p)


def kernel(x, conv1_w, conv1_b, conv2_w, conv2_b,
           fc1_w, fc1_b, fc2_w, fc2_b, fc3_w, fc3_b):
    n = x.shape[0]
    n_pad = ((n + BTILE - 1) // BTILE) * BTILE
    xc = x.astype(jnp.float32).reshape(n, 784)
    if n_pad != n:
        xc = jnp.pad(xc, ((0, n_pad - n), (0, 0)))
    w2r = jnp.pad(conv2_w.reshape(16, 6, 4).transpose(0, 2, 1),
                  ((0, 0), (0, 0), (0, 2))).reshape(16, 32)
    b2c = conv2_b.reshape(16, 1)
    fc1r = (fc1_w.reshape(128, 16, 7, 8)[:, :, :, :7]
            .transpose(0, 2, 3, 1).reshape(128, 784))
    out = _forward(xc, conv1_w, conv1_b, w2r, b2c,
                   fc1r, fc1_b, fc2_w, fc2_b, fc3_w, fc3_b)
    return out[:10, :n].T


# speedup vs baseline: 2.1381x; 1.0163x over previous
import jax
import jax.numpy as jnp
from jax.experimental import pallas as pl
from jax.experimental.pallas import tpu as pltpu

BTILE = 1024
NOUT_PAD = 16


def _lenet_kernel(x_ref, w1_ref, b1_ref, w2r_ref, b2c_ref,
                  w1p_ref, b1p_ref, w2p_ref, b2p_ref, w3p_ref, b3p_ref,
                  o_ref, ps_ref, pb_ref, f2_ref):
    B = o_ref.shape[-1]

    xt = jnp.transpose(x_ref[...], (1, 0))
    xr = xt.reshape(98, 8, B)
    zlane = jnp.zeros((1, B), jnp.float32)

    rows4 = {}

    def prow(t, d4):
        if (t, d4) not in rows4:
            q = 28 * t + d4
            srcs = [xr[(q + 4 * s) // 8, (q + 4 * s) % 8].reshape(1, B)
                    for s in range(7)]
            rows4[(t, d4)] = jnp.concatenate(srcs + [zlane])
        return rows4[(t, d4)]

    zrow = jnp.zeros((8, B), jnp.float32)
    slabs1 = {}

    def slab1(c, d):
        if (c, d) not in slabs1:
            d4, dq = (d - 2) % 4, (d - 2) // 4
            rows = []
            for ii in range(8):
                t = 4 * ii + c - 2
                if 0 <= t < 28:
                    r = prow(t, d4)
                    if dq == -1:
                        r = jnp.pad(r[0:7, :], ((1, 0), (0, 0)))
                    rows.append(r)
                else:
                    rows.append(zrow)
            slabs1[(c, d)] = jnp.stack(rows)
        return slabs1[(c, d)]

    for ch in range(6):
        for al in range(2):
            for be in range(2):
                best = None
                for a in range(2):
                    for b in range(2):
                        acc = None
                        for dh in range(3):
                            for dw in range(3):
                                t = slab1(2 * al + a + dh, 2 * be + b + dw) \
                                    * w1_ref[ch, dh * 3 + dw]
                                acc = t if acc is None else acc + t
                        best = acc if best is None else jnp.maximum(best, acc)
                ps_ref[ch, al, be] = jnp.maximum(best + b1_ref[ch], 0.0)

    for al in range(2):
        for be in range(2):
            for ii in range(8):
                chunk = ps_ref[:, al, be, ii]
                tile = jnp.transpose(chunk, (1, 0, 2))
                tile = jnp.pad(tile, ((0, 0), (0, 2), (0, 0)))
                pb_ref[al * 128 + be * 64 + ii * 8:
                       al * 128 + be * 64 + ii * 8 + 8] = tile

    def pix(i, j):
        return (i % 2) * 128 + (j % 2) * 64 + (i // 2) * 8 + (j // 2)

    w2r = w2r_ref[...]
    b2c = b2c_ref[...]
    for m in range(7):
        for n in range(7):
            best = None
            for g in range(2):
                for h in range(2):
                    v, w = 2 * m + g, 2 * n + h
                    rhs = jnp.stack(
                        [pb_ref[pix(v, w)], pb_ref[pix(v, w + 1)],
                         pb_ref[pix(v + 1, w)], pb_ref[pix(v + 1, w + 1)]]
                    ).reshape(32, B)
                    z = jnp.dot(w2r, rhs,
                                preferred_element_type=jnp.float32)
                    best = z if best is None else jnp.maximum(best, z)
            f2_ref[m * 7 + n] = jnp.maximum(best + b2c, 0.0)

    feats = f2_ref[...].reshape(49 * 16, B)
    h = jnp.dot(w1p_ref[...], feats, preferred_element_type=jnp.float32)
    h = jnp.maximum(h + b1p_ref[...], 0.0)
    h = jnp.dot(w2p_ref[...], h, preferred_element_type=jnp.float32)
    h = jnp.maximum(h + b2p_ref[...], 0.0)
    out = jnp.dot(w3p_ref[...], h, preferred_element_type=jnp.float32)
    o_ref[...] = (out + b3p_ref[...]).astype(o_ref.dtype)


def _forward(xc, c1w, c1b, w2r, b2c, w1p, b1p, w2p, b2p, w3p, b3p):
    n_pad = xc.shape[0]
    nt = n_pad // BTILE
    flops = n_pad * (2 * 6 * 9 * 30 * 30 + 2 * 16 * 32 * 14 * 14
                     + 2 * (784 * 128 + 128 * 128 + 128 * NOUT_PAD))
    bytes_accessed = 4 * (896 * n_pad + w1p.size + w2p.size
                          + w3p.size + NOUT_PAD * n_pad)
    smem = pl.BlockSpec(memory_space=pltpu.MemorySpace.SMEM)
    return pl.pallas_call(
        _lenet_kernel,
        out_shape=jax.ShapeDtypeStruct((NOUT_PAD, n_pad), jnp.float32),
        grid=(nt,),
        in_specs=[
            pl.BlockSpec((BTILE, 784), lambda i: (i, 0)),
            smem, smem,
            pl.BlockSpec((16, 32), lambda i: (0, 0)),
            pl.BlockSpec((16, 1), lambda i: (0, 0)),
            pl.BlockSpec((128, 784), lambda i: (0, 0)),
            pl.BlockSpec((128, 1), lambda i: (0, 0)),
            pl.BlockSpec((128, 128), lambda i: (0, 0)),
            pl.BlockSpec((128, 1), lambda i: (0, 0)),
            pl.BlockSpec((NOUT_PAD, 128), lambda i: (0, 0)),
            pl.BlockSpec((NOUT_PAD, 1), lambda i: (0, 0)),
        ],
        out_specs=pl.BlockSpec((NOUT_PAD, BTILE), lambda i: (0, i)),
        scratch_shapes=[
            pltpu.VMEM((6, 2, 2, 8, 8, BTILE), jnp.float32),
            pltpu.VMEM((256, 8, BTILE), jnp.float32),
            pltpu.VMEM((49, 16, BTILE), jnp.float32),
        ],
        compiler_params=pltpu.CompilerParams(
            dimension_semantics=("parallel",),
            vmem_limit_bytes=64 * 1024 * 1024),
        cost_estimate=pl.CostEstimate(flops=flops, transcendentals=0,
                                      bytes_accessed=bytes_accessed),
    )(xc, c1w, c1b, w2r, b2c, w1p, b1p, w2p, b2p, w3p, b3p)


def kernel(x, conv1_w, conv1_b, conv2_w, conv2_b,
           fc1_w, fc1_b, fc2_w, fc2_b, fc3_w, fc3_b):
    n = x.shape[0]
    n_pad = ((n + BTILE - 1) // BTILE) * BTILE
    xc = x.astype(jnp.float32).reshape(n, 784)
    if n_pad != n:
        xc = jnp.pad(xc, ((0, n_pad - n), (0, 0)))
    w2r = jnp.pad(conv2_w.reshape(16, 6, 4).transpose(0, 2, 1),
                  ((0, 0), (0, 0), (0, 2))).reshape(16, 32)
    b2c = conv2_b.reshape(16, 1)
    fc1r = (fc1_w.reshape(128, 16, 7, 8)[:, :, :, :7]
            .transpose(0, 2, 3, 1).reshape(128, 784))
    out = _forward(xc, conv1_w, conv1_b, w2r, b2c,
                   fc1r, fc1_b, fc2_w, fc2_b, fc3_w, fc3_b)
    return out[:10, :n].T
